# baseline (speedup 1.0000x reference)
import jax
import jax.numpy as jnp
from jax.experimental import pallas as pl
from jax.experimental.pallas import tpu as pltpu

_B, _N, _E = 32, 100, 400
_NP = 104
_R = _B * _NP
_C = 4
_CR = _R // _C
_CG = _B // _C
_HID, _HEADS, _HD, _EDIM, _FF = 512, 8, 64, 64, 2048
_LAT, _LAYERS, _VOCAB, _MAXN = 1024, 4, 10000, 100
_F32 = jnp.float32
_BF16 = jnp.bfloat16


def _gelu(x):
    return 0.5 * x * (1.0 + jax.lax.erf(x * (2.0 ** -0.5)))


def _ln2d(x, g, b, eps=1e-5):
    m = jnp.mean(x, -1, keepdims=True)
    v = jnp.mean((x - m) ** 2, -1, keepdims=True)
    return (x - m) / jnp.sqrt(v + eps) * g + b


def _dot(a, b):
    return jnp.dot(a, b, preferred_element_type=_F32)


def _dotT(a, w):
    return jax.lax.dot_general(a, w, (((1,), (1,)), ((), ())),
                               preferred_element_type=_F32)


def _dotL(m, x):
    return jax.lax.dot_general(m, x, (((0,), (0,)), ((), ())),
                               preferred_element_type=_F32)


def _fwd_kernel(tool_sref,
                tier_ref, scope_ref, emb_hbm, pos_ref, ttab_ref, stab_ref,
                fusW, fusb, argW, argb, ing, inb,
                srcR_ref, tgtR_ref,
                Wq, bq, Wk, bk, Wv, bv, Z1, Z2, cbL, Wout, bout,
                n1g, n1b, W1_hbm, b1, W2_hbm, b2, n2g, n2b,
                poolW, poolb, pj1, pj1b, pj2, pj2b, png, pnb,
                out_ref, x_all, acc, qs, ks_, vs_, ess, w1s, w2s, sem, semw):
    def _zpad(b, c):
        x_all[b, pl.ds(_N, _NP - _N), :] = jnp.zeros((_NP - _N, _HID), _F32)
        return c

    jax.lax.fori_loop(0, _B, _zpad, 0)

    def _start(i, c):
        b = jax.lax.div(i, _N)
        n = jax.lax.rem(i, _N)
        pltpu.make_async_copy(emb_hbm.at[tool_sref[b, n]], x_all.at[b, n], sem).start()
        return c

    jax.lax.fori_loop(0, _B * _N, _start, 0)

    for c in range(_C):
        ch = slice(c * _CR, (c + 1) * _CR)
        tier = tier_ref[:, ch]
        scope = scope_ref[:, ch]
        toh = (tier == jax.lax.broadcasted_iota(jnp.int32, (3, _CR), 0)).astype(_BF16)
        soh = (scope == jax.lax.broadcasted_iota(jnp.int32, (10, _CR), 0)).astype(_BF16)
        te = _dotL(toh, ttab_ref[:])
        se = _dotL(soh, stab_ref[:])
        base = _dotT(jnp.concatenate([te, se], axis=1).astype(_BF16), fusW[:]) + fusb[0]
        base = base.reshape(_CG, _NP, _HID) + pos_ref[:][None]
        qs[ch, :] = base.reshape(_CR, _HID).astype(_BF16)

    def _wait(i, c):
        pltpu.make_async_copy(emb_hbm.at[0], x_all.at[0, 0], sem).wait()
        return c

    jax.lax.fori_loop(0, _B * _N, _wait, 0)

    for c in range(_C):
        cg = slice(c * _CG, (c + 1) * _CG)
        ch = slice(c * _CR, (c + 1) * _CR)
        v0 = (x_all[cg].reshape(_CR, _HID) + qs[ch, :].astype(_F32)).astype(_BF16)
        x1 = _gelu(_dotT(v0, argW[:]) + argb[0])
        x_all[cg] = _ln2d(x1, ing[0], inb[0]).reshape(_CG, _NP, _HID)

    for g0 in range(_B):
        acc[g0 * _NP + _N:g0 * _NP + _NP, :] = jnp.zeros((_NP - _N, _HID), _BF16)

    pool_m = ((jax.lax.broadcasted_iota(jnp.int32, (_HID, _HEADS), 0) // _HD
               == jax.lax.broadcasted_iota(jnp.int32, (_HID, _HEADS), 1))
              .astype(_F32) * (_HD ** -0.5)).astype(_BF16)
    exp_m = (jax.lax.broadcasted_iota(jnp.int32, (_HEADS, _HID), 1) // _HD
             == jax.lax.broadcasted_iota(jnp.int32, (_HEADS, _HID), 0)).astype(_BF16)

    for l in range(_LAYERS):
        pltpu.make_async_copy(W1_hbm.at[l], w1s, semw).start()
        pltpu.make_async_copy(W2_hbm.at[l], w2s, semw).start()

        for c in range(_C):
            cg = slice(c * _CG, (c + 1) * _CG)
            ch = slice(c * _CR, (c + 1) * _CR)
            hb = _ln2d(x_all[cg].reshape(_CR, _HID), n1g[l], n1b[l]).astype(_BF16)
            qs[ch, :] = (_dotT(hb, Wq[l]) + bq[l]).astype(_BF16)
            ks_[ch, :] = (_dotT(hb, Wk[l]) + bk[l]).astype(_BF16)
            vs_[ch, :] = (_dotT(hb, Wv[l]) + bv[l]).astype(_BF16)
            ess[ch, :_HEADS] = _dotT(hb, Z1[l]).astype(_BF16)
            ess[ch, _HEADS:] = _dotT(hb, Z2[l]).astype(_BF16)
        cb = cbL[l]

        def _gat(g, c):
            r0 = pl.multiple_of(g * _NP, 8)
            srcR = srcR_ref[g]
            tgtR = tgtR_ref[g]
            src_NE = (srcR == jax.lax.broadcasted_iota(
                jnp.int32, (_N, _E), 0)).astype(_BF16)
            tgt_NE = (tgtR == jax.lax.broadcasted_iota(
                jnp.int32, (_N, _E), 0)).astype(_BF16)

            qg = qs[pl.ds(r0, _N), :]
            kg = ks_[pl.ds(r0, _N), :]
            vg = vs_[pl.ds(r0, _N), :]
            esg_s = ess[pl.ds(r0, _N), :_HEADS]
            esg_t = ess[pl.ds(r0, _N), _HEADS:]

            gt = _dotL(tgt_NE, qg)
            ks = _dotL(src_NE, kg)
            vs = _dotL(src_NE, vg)

            scores = _dot((gt * ks).astype(_BF16), pool_m)
            scores = scores + _dotL(src_NE, esg_s)
            scores = scores + _dotL(tgt_NE, esg_t) + cb

            gmax = jnp.max(scores)
            ex = jnp.exp(scores - gmax)
            den = _dot(tgt_NE, ex.astype(_BF16))
            den_t = _dotL(tgt_NE, den.astype(_BF16))
            w = ex / den_t
            wf = _dot(w.astype(_BF16), exp_m)
            agg = _dot(tgt_NE, (wf * vs).astype(_BF16))
            acc[pl.ds(r0, _N), :] = agg.astype(_BF16)
            return c

        jax.lax.fori_loop(0, _B, _gat, 0)

        pltpu.make_async_copy(W1_hbm.at[l], w1s, semw).wait()
        pltpu.make_async_copy(W2_hbm.at[l], w2s, semw).wait()

        for c in range(_C):
            cg = slice(c * _CG, (c + 1) * _CG)
            ch = slice(c * _CR, (c + 1) * _CR)
            xc = x_all[cg].reshape(_CR, _HID)
            x2 = xc + _dotT(acc[ch, :], Wout[l]) + bout[l]
            h2 = _ln2d(x2, n2g[l], n2b[l]).astype(_BF16)
            f1 = _gelu(_dotT(h2, w1s[:]) + b1[l]).astype(_BF16)
            f2 = _dotT(f1, w2s[:]) + b2[l]
            x_all[cg] = (x2 + f2).reshape(_CG, _NP, _HID)

    x3 = x_all[:]
    s = jnp.sum(x3 * poolW[0][None, None, :], axis=-1) + poolb[0, 0]
    valid = jax.lax.broadcasted_iota(jnp.int32, (_B, _NP), 1) < _N
    s = jnp.where(valid, s, -1e30)
    s = s - jnp.max(s, axis=1, keepdims=True)
    es = jnp.exp(s)
    a = es / jnp.sum(es, axis=1, keepdims=True)
    pooled = jnp.sum(a[:, :, None] * x3, axis=1)
    z = _gelu(_dotT(pooled.astype(_BF16), pj1[:]) + pj1b[0])
    z = _dotT(z.astype(_BF16), pj2[:]) + pj2b[0]
    out_ref[:] = _ln2d(z, png[0], pnb[0])


def kernel(params, tool_indices, tier_indices, scope_indices, edge_index):
    p = params
    tool_idx = tool_indices.astype(jnp.int32)
    pad_w = ((0, 0), (0, _NP - _N))
    tier_r = jnp.pad(tier_indices.astype(jnp.int32), pad_w).reshape(1, _R)
    scope_r = jnp.pad(scope_indices.astype(jnp.int32), pad_w).reshape(1, _R)
    ei = edge_index.astype(jnp.int32)
    srcR = ei[:, :, 0][:, None, :]
    tgtR = ei[:, :, 1][:, None, :]

    pos = jnp.pad(p['pos_emb'][:_N], ((0, _NP - _N), (0, 0)))
    Z = jnp.einsum('lhe,leo->lho', p['Wea'], p['Wep'])
    cbL = (jnp.einsum('le,lhe->lh', p['bep'], p['Wea'])
           + p['bea'])[:, None, :]

    def row(v):
        return v.reshape(1, -1)

    def rowl(v):
        return v[:, None, :]

    in_arrays = [
        tier_r, scope_r, p['tool_emb'], pos, p['tier_emb'], p['scope_emb'],
        p['fusion_W'].astype(_BF16), row(p['fusion_b']),
        p['arg_W'].astype(_BF16), row(p['arg_b']),
        row(p['in_g']), row(p['in_b']),
        srcR, tgtR,
        p['Wq'].astype(_BF16), rowl(p['bq']),
        p['Wk'].astype(_BF16), rowl(p['bk']),
        p['Wv'].astype(_BF16), rowl(p['bv']),
        Z[:, :, :_HID].astype(_BF16), Z[:, :, _HID:].astype(_BF16), cbL,
        p['Wout'].astype(_BF16), rowl(p['bout']),
        rowl(p['n1g']), rowl(p['n1b']),
        p['W1'].astype(_BF16), rowl(p['b1']),
        p['W2'].astype(_BF16), rowl(p['b2']),
        rowl(p['n2g']), rowl(p['n2b']),
        row(p['pool_W'][0]), p['pool_b'].reshape(1, 1),
        p['pj1_W'].astype(_BF16), row(p['pj1_b']),
        p['pj2_W'].astype(_BF16), row(p['pj2_b']),
        row(p['pjn_g']), row(p['pjn_b']),
    ]
    _any_idx = {2, 27, 29}
    in_specs = [pl.BlockSpec(memory_space=pl.ANY) if i in _any_idx
                else pl.BlockSpec(a.shape, (lambda nd: lambda g, *_: (0,) * nd)(a.ndim))
                for i, a in enumerate(in_arrays)]

    grid_spec = pltpu.PrefetchScalarGridSpec(
        num_scalar_prefetch=1,
        grid=(1,),
        in_specs=in_specs,
        out_specs=pl.BlockSpec((_B, _LAT), lambda g, *_: (0, 0)),
        scratch_shapes=[pltpu.VMEM((_B, _NP, _HID), _F32),
                        pltpu.VMEM((_R, _HID), _BF16),
                        pltpu.VMEM((_R, _HID), _BF16),
                        pltpu.VMEM((_R, _HID), _BF16),
                        pltpu.VMEM((_R, _HID), _BF16),
                        pltpu.VMEM((_R, 2 * _HEADS), _BF16),
                        pltpu.VMEM((_FF, _HID), _BF16),
                        pltpu.VMEM((_HID, _FF), _BF16),
                        pltpu.SemaphoreType.DMA,
                        pltpu.SemaphoreType.DMA],
    )
    out = pl.pallas_call(
        _fwd_kernel,
        grid_spec=grid_spec,
        out_shape=jax.ShapeDtypeStruct((_B, _LAT), _F32),
    )(tool_idx, *in_arrays)
    return out
